# baseline (speedup 1.0000x reference)
import functools

import jax
import jax.numpy as jnp
from jax import lax
from jax.experimental import pallas as pl
from jax.experimental.pallas import tpu as pltpu
from jax.experimental.pallas import tpu_sc as plsc

TOPK = 2
BLK = 256
RBLK = 512
NC, NS, LANES = 2, 16, 16
NW = NC * NS

_SC_MESH = dict(core_axis_name="c", subcore_axis_name="s",
                num_cores=NC, num_subcores=NS)


def _router_body(x_ref, gw_ref, logits_ref, topw_ref, topi_ref):
    x = x_ref[...]
    logits = lax.dot_general(x, gw_ref[...], (((1,), (1,)), ((), ())),
                             preferred_element_type=jnp.float32)
    logits_ref[...] = logits
    e = logits.shape[1]
    m = jnp.max(logits, axis=1, keepdims=True)
    p = jnp.exp(logits - m)
    probs = p / jnp.sum(p, axis=1, keepdims=True)
    iota = lax.broadcasted_iota(jnp.int32, probs.shape, 1)
    m1 = jnp.max(probs, axis=1, keepdims=True)
    i1 = jnp.min(jnp.where(probs == m1, iota, e), axis=1, keepdims=True)
    probs2 = jnp.where(iota == i1, -jnp.inf, probs)
    m2 = jnp.max(probs2, axis=1, keepdims=True)
    i2 = jnp.min(jnp.where(probs2 == m2, iota, e), axis=1, keepdims=True)
    topw_ref[...] = jnp.concatenate([m1, m2], axis=1)
    topi_ref[...] = jnp.concatenate([i1, i2], axis=1)


def _mlp_body(be_ref, bv_ref, xs_ref, wg_ref, wu_ref, wd_ref, w_ref, out_ref):
    i = pl.program_id(0)

    @pl.when(bv_ref[i] != 0)
    def _():
        xb = xs_ref[...].astype(jnp.float32)
        g = lax.dot_general(xb, wg_ref[0], (((1,), (1,)), ((), ())),
                            preferred_element_type=jnp.float32)
        u = lax.dot_general(xb, wu_ref[0], (((1,), (1,)), ((), ())),
                            preferred_element_type=jnp.float32)
        hv = (g * jax.nn.sigmoid(g)) * u
        o = lax.dot_general(hv, wd_ref[0], (((1,), (1,)), ((), ())),
                            preferred_element_type=jnp.float32)
        out_ref[...] = o * w_ref[...]


def _sc_gather(x, row_token, npad):
    t, d = x.shape
    b_per_w = npad // NW
    ch = 40
    n = b_per_w // ch
    mesh = plsc.VectorSubcoreMesh(**_SC_MESH)

    @functools.partial(
        pl.kernel, mesh=mesh,
        out_type=jax.ShapeDtypeStruct((npad, d), jnp.int32),
        scratch_types=[pltpu.VMEM((ch,), jnp.int32),
                       pltpu.VMEM((ch,), jnp.int32),
                       pltpu.VMEM((ch, d), jnp.int32),
                       pltpu.VMEM((ch, d), jnp.int32),
                       pltpu.SemaphoreType.DMA,
                       pltpu.SemaphoreType.DMA,
                       pltpu.SemaphoreType.DMA,
                       pltpu.SemaphoreType.DMA],
    )
    def gather_k(x_hbm, tok_hbm, xs_hbm, idx0, idx1, rows0, rows1,
                 g0, g1, w0, w1):
        idx, rows, gs, ws = [idx0, idx1], [rows0, rows1], [g0, g1], [w0, w1]
        wid = lax.axis_index("s") * NC + lax.axis_index("c")
        base = wid * b_per_w
        gd, wd = [None] * n, [None] * n
        pltpu.sync_copy(tok_hbm.at[pl.ds(base, ch)], idx[0])
        gd[0] = pltpu.async_copy(x_hbm.at[idx[0]], rows[0], gs[0])
        for c in range(n):
            cur = c & 1
            nxt = 1 - cur
            if c + 1 < n:
                pltpu.sync_copy(tok_hbm.at[pl.ds(base + (c + 1) * ch, ch)],
                                idx[nxt])
                if c >= 1:
                    wd[c - 1].wait()
                gd[c + 1] = pltpu.async_copy(x_hbm.at[idx[nxt]], rows[nxt],
                                             gs[nxt])
            gd[c].wait()
            wd[c] = pltpu.async_copy(rows[cur],
                                     xs_hbm.at[pl.ds(base + c * ch, ch)],
                                     ws[cur])
        if n >= 2:
            wd[n - 2].wait()
        wd[n - 1].wait()

    return gather_k(x, row_token)


def _sc_combine(wout, pos0, pos1):
    t = pos0.shape[0]
    d = wout.shape[1]
    t_per_w = t // NW
    ch = 8
    n = t_per_w // ch
    mesh = plsc.VectorSubcoreMesh(**_SC_MESH)

    @functools.partial(
        pl.kernel, mesh=mesh,
        out_type=jax.ShapeDtypeStruct((t, d), jnp.float32),
        scratch_types=[pltpu.VMEM((ch,), jnp.int32),
                       pltpu.VMEM((ch,), jnp.int32),
                       pltpu.VMEM((ch,), jnp.int32),
                       pltpu.VMEM((ch,), jnp.int32),
                       pltpu.VMEM((ch, d), jnp.float32),
                       pltpu.VMEM((ch, d), jnp.float32),
                       pltpu.VMEM((ch, d), jnp.float32),
                       pltpu.VMEM((ch, d), jnp.float32),
                       pltpu.SemaphoreType.DMA,
                       pltpu.SemaphoreType.DMA,
                       pltpu.SemaphoreType.DMA,
                       pltpu.SemaphoreType.DMA],
    )
    def combine_k(wout_hbm, p0_hbm, p1_hbm, out_hbm,
                  p0a, p0b, p1a, p1b, r0a, r0b, r1a, r1b, g0, g1, w0, w1):
        p0, p1 = [p0a, p0b], [p1a, p1b]
        r0, r1, gs, ws = [r0a, r0b], [r1a, r1b], [g0, g1], [w0, w1]
        wid = lax.axis_index("s") * NC + lax.axis_index("c")
        base = wid * t_per_w
        g0d, g1d, wd = [None] * n, [None] * n, [None] * n
        pltpu.sync_copy(p0_hbm.at[pl.ds(base, ch)], p0[0])
        pltpu.sync_copy(p1_hbm.at[pl.ds(base, ch)], p1[0])
        g0d[0] = pltpu.async_copy(wout_hbm.at[p0[0]], r0[0], gs[0])
        g1d[0] = pltpu.async_copy(wout_hbm.at[p1[0]], r1[0], gs[0])
        for c in range(n):
            cur = c & 1
            nxt = 1 - cur
            if c + 1 < n:
                off_n = base + (c + 1) * ch
                pltpu.sync_copy(p0_hbm.at[pl.ds(off_n, ch)], p0[nxt])
                pltpu.sync_copy(p1_hbm.at[pl.ds(off_n, ch)], p1[nxt])
                if c >= 1:
                    wd[c - 1].wait()
                g0d[c + 1] = pltpu.async_copy(wout_hbm.at[p0[nxt]], r0[nxt],
                                              gs[nxt])
                g1d[c + 1] = pltpu.async_copy(wout_hbm.at[p1[nxt]], r1[nxt],
                                              gs[nxt])
            g0d[c].wait()
            g1d[c].wait()
            for r in range(ch):
                def add_body(ci, _, r=r, cur=cur):
                    sl = pl.ds(ci * LANES, LANES)
                    r0[cur][r, sl] = r0[cur][r, sl] + r1[cur][r, sl]
                    return 0
                lax.fori_loop(0, d // LANES, add_body, 0)
            wd[c] = pltpu.async_copy(r0[cur],
                                     out_hbm.at[pl.ds(base + c * ch, ch)],
                                     ws[cur])
        if n >= 2:
            wd[n - 2].wait()
        wd[n - 1].wait()

    return combine_k(wout, pos0, pos1)


def kernel(hidden_states, gate_w, Wg, Wu, Wd):
    b, s, d = hidden_states.shape
    e, h, _ = Wg.shape
    t = b * s
    p = t * TOPK
    nb = (p + e * (BLK - 1) + BLK - 1) // BLK
    npad = nb * BLK

    x = hidden_states.reshape(t, d)

    logits, topw, topi = pl.pallas_call(
        _router_body,
        grid=(t // RBLK,),
        in_specs=[pl.BlockSpec((RBLK, d), lambda i: (i, 0)),
                  pl.BlockSpec((e, d), lambda i: (0, 0))],
        out_specs=[pl.BlockSpec((RBLK, e), lambda i: (i, 0)),
                   pl.BlockSpec((RBLK, TOPK), lambda i: (i, 0)),
                   pl.BlockSpec((RBLK, TOPK), lambda i: (i, 0))],
        out_shape=[jax.ShapeDtypeStruct((t, e), jnp.float32),
                   jax.ShapeDtypeStruct((t, TOPK), jnp.float32),
                   jax.ShapeDtypeStruct((t, TOPK), jnp.int32)],
    )(x, gate_w)

    pair_e = topi.reshape(-1)
    pair_w = topw.reshape(-1)
    sort_idx = jnp.argsort(pair_e, stable=True)
    se = pair_e[sort_idx]
    counts = jnp.zeros((e,), jnp.int32).at[pair_e].add(1)
    pad_counts = ((counts + BLK - 1) // BLK) * BLK
    ends = jnp.cumsum(pad_counts)
    pad_off = ends - pad_counts
    un_off = jnp.cumsum(counts) - counts
    rank = jnp.arange(p, dtype=jnp.int32) - un_off[se]
    dest = pad_off[se] + rank
    row_token = jnp.zeros((npad,), jnp.int32).at[dest].set(
        (sort_idx // TOPK).astype(jnp.int32))
    row_w = jnp.zeros((npad,), jnp.float32).at[dest].set(pair_w[sort_idx])
    pos = jnp.zeros((p,), jnp.int32).at[sort_idx].set(dest)
    pos0 = pos[0::TOPK]
    pos1 = pos[1::TOPK]
    total = ends[-1]
    bstart = jnp.arange(nb, dtype=jnp.int32) * BLK
    block_expert = jnp.minimum(
        jnp.searchsorted(ends, bstart, side="right").astype(jnp.int32), e - 1)
    block_valid = (bstart < total).astype(jnp.int32)

    x_pk = lax.bitcast_convert_type(
        x.astype(jnp.bfloat16).reshape(t, d // 2, 2), jnp.int32)
    xs_pk = _sc_gather(x_pk, row_token, npad)
    xs = lax.bitcast_convert_type(xs_pk, jnp.bfloat16).reshape(npad, d)

    grid_spec = pltpu.PrefetchScalarGridSpec(
        num_scalar_prefetch=2,
        grid=(nb,),
        in_specs=[
            pl.BlockSpec((BLK, d), lambda i, be, bv: (i, 0)),
            pl.BlockSpec((1, h, d), lambda i, be, bv: (be[i], 0, 0)),
            pl.BlockSpec((1, h, d), lambda i, be, bv: (be[i], 0, 0)),
            pl.BlockSpec((1, d, h), lambda i, be, bv: (be[i], 0, 0)),
            pl.BlockSpec((BLK, 1), lambda i, be, bv: (i, 0)),
        ],
        out_specs=pl.BlockSpec((BLK, d), lambda i, be, bv: (i, 0)),
    )
    wout = pl.pallas_call(
        _mlp_body,
        grid_spec=grid_spec,
        out_shape=jax.ShapeDtypeStruct((npad, d), jnp.float32),
    )(block_expert, block_valid, xs, Wg, Wu, Wd, row_w.reshape(npad, 1))

    final = _sc_combine(wout, pos0, pos1)

    return final.reshape(b, s, d), logits


# speedup vs baseline: 2.0804x; 2.0804x over previous
import functools

import jax
import jax.numpy as jnp
from jax import lax
from jax.experimental import pallas as pl
from jax.experimental.pallas import tpu as pltpu
from jax.experimental.pallas import tpu_sc as plsc

TOPK = 2
BLK = 256
RBLK = 512
NC, NS, LANES = 2, 16, 16
NW = NC * NS

_SC_MESH = dict(core_axis_name="c", subcore_axis_name="s",
                num_cores=NC, num_subcores=NS)


def _router_body(x_ref, gw_ref, logits_ref, topw_ref, topi_ref, xpk_ref):
    x = x_ref[...]
    logits = lax.dot_general(x, gw_ref[...], (((1,), (1,)), ((), ())),
                             preferred_element_type=jnp.float32)
    logits_ref[...] = logits
    d2 = x.shape[1] // 2
    lo = x[:, :d2].astype(jnp.bfloat16).astype(jnp.float32)
    hi = x[:, d2:].astype(jnp.bfloat16).astype(jnp.float32)
    lo_u = lax.bitcast_convert_type(lo, jnp.uint32) >> 16
    hi_u = lax.bitcast_convert_type(hi, jnp.uint32) & jnp.uint32(0xFFFF0000)
    xpk_ref[...] = lax.bitcast_convert_type(hi_u | lo_u, jnp.int32)
    e = logits.shape[1]
    m = jnp.max(logits, axis=1, keepdims=True)
    p = jnp.exp(logits - m)
    probs = p / jnp.sum(p, axis=1, keepdims=True)
    iota = lax.broadcasted_iota(jnp.int32, probs.shape, 1)
    m1 = jnp.max(probs, axis=1, keepdims=True)
    i1 = jnp.min(jnp.where(probs == m1, iota, e), axis=1, keepdims=True)
    probs2 = jnp.where(iota == i1, -jnp.inf, probs)
    m2 = jnp.max(probs2, axis=1, keepdims=True)
    i2 = jnp.min(jnp.where(probs2 == m2, iota, e), axis=1, keepdims=True)
    topw_ref[...] = jnp.concatenate([m1, m2], axis=1)
    topi_ref[...] = jnp.concatenate([i1, i2], axis=1)


def _mlp_body(be_ref, bv_ref, xs_ref, wg_ref, wu_ref, wd_ref, w_ref, out_ref):
    i = pl.program_id(0)

    @pl.when(bv_ref[i] != 0)
    def _():
        xi = lax.bitcast_convert_type(xs_ref[...], jnp.uint32)
        x_lo = lax.bitcast_convert_type(xi << 16, jnp.float32)
        x_hi = lax.bitcast_convert_type(xi & jnp.uint32(0xFFFF0000),
                                        jnp.float32)
        d2 = xi.shape[1]
        dn = (((1,), (1,)), ((), ()))
        wg, wu, wd = wg_ref[0], wu_ref[0], wd_ref[0]
        g = (lax.dot_general(x_lo, wg[:, :d2], dn,
                             preferred_element_type=jnp.float32)
             + lax.dot_general(x_hi, wg[:, d2:], dn,
                               preferred_element_type=jnp.float32))
        u = (lax.dot_general(x_lo, wu[:, :d2], dn,
                             preferred_element_type=jnp.float32)
             + lax.dot_general(x_hi, wu[:, d2:], dn,
                               preferred_element_type=jnp.float32))
        hv = (g * jax.nn.sigmoid(g)) * u
        o = lax.dot_general(hv, wd, dn, preferred_element_type=jnp.float32)
        out_ref[...] = o * w_ref[...]


def _sc_gather(x, row_token, npad):
    t, d = x.shape
    b_per_w = npad // NW
    ch = 40
    n = b_per_w // ch
    mesh = plsc.VectorSubcoreMesh(**_SC_MESH)

    @functools.partial(
        pl.kernel, mesh=mesh,
        out_type=jax.ShapeDtypeStruct((npad, d), jnp.int32),
        scratch_types=[pltpu.VMEM((ch,), jnp.int32),
                       pltpu.VMEM((ch,), jnp.int32),
                       pltpu.VMEM((ch, d), jnp.int32),
                       pltpu.VMEM((ch, d), jnp.int32),
                       pltpu.SemaphoreType.DMA,
                       pltpu.SemaphoreType.DMA,
                       pltpu.SemaphoreType.DMA,
                       pltpu.SemaphoreType.DMA],
    )
    def gather_k(x_hbm, tok_hbm, xs_hbm, idx0, idx1, rows0, rows1,
                 g0, g1, w0, w1):
        idx, rows, gs, ws = [idx0, idx1], [rows0, rows1], [g0, g1], [w0, w1]
        wid = lax.axis_index("s") * NC + lax.axis_index("c")
        base = wid * b_per_w
        gd, wd = [None] * n, [None] * n
        pltpu.sync_copy(tok_hbm.at[pl.ds(base, ch)], idx[0])
        gd[0] = pltpu.async_copy(x_hbm.at[idx[0]], rows[0], gs[0])
        for c in range(n):
            cur = c & 1
            nxt = 1 - cur
            if c + 1 < n:
                pltpu.sync_copy(tok_hbm.at[pl.ds(base + (c + 1) * ch, ch)],
                                idx[nxt])
                if c >= 1:
                    wd[c - 1].wait()
                gd[c + 1] = pltpu.async_copy(x_hbm.at[idx[nxt]], rows[nxt],
                                             gs[nxt])
            gd[c].wait()
            wd[c] = pltpu.async_copy(rows[cur],
                                     xs_hbm.at[pl.ds(base + c * ch, ch)],
                                     ws[cur])
        if n >= 2:
            wd[n - 2].wait()
        wd[n - 1].wait()

    return gather_k(x, row_token)


def _sc_combine(wout, pos0, pos1):
    t = pos0.shape[0]
    d = wout.shape[1]
    t_per_w = t // NW
    ch = 8
    n = t_per_w // ch
    mesh = plsc.VectorSubcoreMesh(**_SC_MESH)

    @functools.partial(
        pl.kernel, mesh=mesh,
        out_type=jax.ShapeDtypeStruct((t, d), jnp.float32),
        scratch_types=[pltpu.VMEM((ch,), jnp.int32),
                       pltpu.VMEM((ch,), jnp.int32),
                       pltpu.VMEM((ch,), jnp.int32),
                       pltpu.VMEM((ch,), jnp.int32),
                       pltpu.VMEM((ch, d), jnp.float32),
                       pltpu.VMEM((ch, d), jnp.float32),
                       pltpu.VMEM((ch, d), jnp.float32),
                       pltpu.VMEM((ch, d), jnp.float32),
                       pltpu.SemaphoreType.DMA,
                       pltpu.SemaphoreType.DMA,
                       pltpu.SemaphoreType.DMA,
                       pltpu.SemaphoreType.DMA],
    )
    def combine_k(wout_hbm, p0_hbm, p1_hbm, out_hbm,
                  p0a, p0b, p1a, p1b, r0a, r0b, r1a, r1b, g0, g1, w0, w1):
        p0, p1 = [p0a, p0b], [p1a, p1b]
        r0, r1, gs, ws = [r0a, r0b], [r1a, r1b], [g0, g1], [w0, w1]
        wid = lax.axis_index("s") * NC + lax.axis_index("c")
        base = wid * t_per_w
        g0d, g1d, wd = [None] * n, [None] * n, [None] * n
        pltpu.sync_copy(p0_hbm.at[pl.ds(base, ch)], p0[0])
        pltpu.sync_copy(p1_hbm.at[pl.ds(base, ch)], p1[0])
        g0d[0] = pltpu.async_copy(wout_hbm.at[p0[0]], r0[0], gs[0])
        g1d[0] = pltpu.async_copy(wout_hbm.at[p1[0]], r1[0], gs[0])
        for c in range(n):
            cur = c & 1
            nxt = 1 - cur
            if c + 1 < n:
                off_n = base + (c + 1) * ch
                pltpu.sync_copy(p0_hbm.at[pl.ds(off_n, ch)], p0[nxt])
                pltpu.sync_copy(p1_hbm.at[pl.ds(off_n, ch)], p1[nxt])
                if c >= 1:
                    wd[c - 1].wait()
                g0d[c + 1] = pltpu.async_copy(wout_hbm.at[p0[nxt]], r0[nxt],
                                              gs[nxt])
                g1d[c + 1] = pltpu.async_copy(wout_hbm.at[p1[nxt]], r1[nxt],
                                              gs[nxt])
            g0d[c].wait()
            g1d[c].wait()
            for r in range(ch):
                def add_body(ci, _, r=r, cur=cur):
                    sl = pl.ds(ci * LANES, LANES)
                    r0[cur][r, sl] = r0[cur][r, sl] + r1[cur][r, sl]
                    return 0
                lax.fori_loop(0, d // LANES, add_body, 0)
            wd[c] = pltpu.async_copy(r0[cur],
                                     out_hbm.at[pl.ds(base + c * ch, ch)],
                                     ws[cur])
        if n >= 2:
            wd[n - 2].wait()
        wd[n - 1].wait()

    return combine_k(wout, pos0, pos1)


def kernel(hidden_states, gate_w, Wg, Wu, Wd):
    b, s, d = hidden_states.shape
    e, h, _ = Wg.shape
    t = b * s
    p = t * TOPK
    nb = (p + e * (BLK - 1) + BLK - 1) // BLK
    npad = nb * BLK

    x = hidden_states.reshape(t, d)

    logits, topw, topi, x_pk = pl.pallas_call(
        _router_body,
        grid=(t // RBLK,),
        in_specs=[pl.BlockSpec((RBLK, d), lambda i: (i, 0)),
                  pl.BlockSpec((e, d), lambda i: (0, 0))],
        out_specs=[pl.BlockSpec((RBLK, e), lambda i: (i, 0)),
                   pl.BlockSpec((RBLK, TOPK), lambda i: (i, 0)),
                   pl.BlockSpec((RBLK, TOPK), lambda i: (i, 0)),
                   pl.BlockSpec((RBLK, d // 2), lambda i: (i, 0))],
        out_shape=[jax.ShapeDtypeStruct((t, e), jnp.float32),
                   jax.ShapeDtypeStruct((t, TOPK), jnp.float32),
                   jax.ShapeDtypeStruct((t, TOPK), jnp.int32),
                   jax.ShapeDtypeStruct((t, d // 2), jnp.int32)],
    )(x, gate_w)

    pair_e = topi.reshape(-1)
    pair_w = topw.reshape(-1)
    sort_idx = jnp.argsort(pair_e, stable=True)
    se = pair_e[sort_idx]
    counts = jnp.zeros((e,), jnp.int32).at[pair_e].add(1)
    pad_counts = ((counts + BLK - 1) // BLK) * BLK
    ends = jnp.cumsum(pad_counts)
    pad_off = ends - pad_counts
    un_off = jnp.cumsum(counts) - counts
    rank = jnp.arange(p, dtype=jnp.int32) - un_off[se]
    dest = pad_off[se] + rank
    row_token = jnp.zeros((npad,), jnp.int32).at[dest].set(
        (sort_idx // TOPK).astype(jnp.int32))
    row_w = jnp.zeros((npad,), jnp.float32).at[dest].set(pair_w[sort_idx])
    pos = jnp.zeros((p,), jnp.int32).at[sort_idx].set(dest)
    pos0 = pos[0::TOPK]
    pos1 = pos[1::TOPK]
    total = ends[-1]
    bstart = jnp.arange(nb, dtype=jnp.int32) * BLK
    block_expert = jnp.minimum(
        jnp.searchsorted(ends, bstart, side="right").astype(jnp.int32), e - 1)
    block_valid = (bstart < total).astype(jnp.int32)

    xs = _sc_gather(x_pk, row_token, npad)

    grid_spec = pltpu.PrefetchScalarGridSpec(
        num_scalar_prefetch=2,
        grid=(nb,),
        in_specs=[
            pl.BlockSpec((BLK, d // 2), lambda i, be, bv: (i, 0)),
            pl.BlockSpec((1, h, d), lambda i, be, bv: (be[i], 0, 0)),
            pl.BlockSpec((1, h, d), lambda i, be, bv: (be[i], 0, 0)),
            pl.BlockSpec((1, d, h), lambda i, be, bv: (be[i], 0, 0)),
            pl.BlockSpec((BLK, 1), lambda i, be, bv: (i, 0)),
        ],
        out_specs=pl.BlockSpec((BLK, d), lambda i, be, bv: (i, 0)),
    )
    wout = pl.pallas_call(
        _mlp_body,
        grid_spec=grid_spec,
        out_shape=jax.ShapeDtypeStruct((npad, d), jnp.float32),
    )(block_expert, block_valid, xs, Wg, Wu, Wd, row_w.reshape(npad, 1))

    final = _sc_combine(wout, pos0, pos1)

    return final.reshape(b, s, d), logits
